# baseline (speedup 1.0000x reference)
import functools

import jax
import jax.numpy as jnp
from jax import lax
from jax.experimental import pallas as pl
from jax.experimental.pallas import tpu as pltpu
from jax.experimental.pallas import tpu_sc as plsc

BATCH = 16384
D = 128
NC = 2
NS = 16
NW = NC * NS
BPW = BATCH // NW
CH = 128
NCH = BPW // CH
NBUF = 6
WB = 2 * CH

_mesh = plsc.VectorSubcoreMesh(core_axis_name="c", subcore_axis_name="s")


@functools.partial(
    pl.kernel,
    mesh=_mesh,
    out_type=(
        jax.ShapeDtypeStruct((BATCH, D), jnp.float32),
        jax.ShapeDtypeStruct((BATCH, D), jnp.float32),
    ),
    scratch_types=[
        pltpu.VMEM((NCH, CH), jnp.int32),
        pltpu.VMEM((NCH, CH), jnp.int32),
        pltpu.VMEM((NBUF * CH, D), jnp.float32),
        pltpu.SemaphoreType.DMA((2,)),
        pltpu.SemaphoreType.DMA((8,)),
        pltpu.SemaphoreType.DMA((4,)),
    ],
)
def _sc_gather(uids, iids, utab, itab, uout, iout, uidx_v, iidx_v,
               bufs, isem, gsem, wsem):
    wid = lax.axis_index("s") * NC + lax.axis_index("c")
    base = wid * BPW

    ldu = pltpu.async_copy(uids.at[wid], uidx_v, isem.at[0])
    ldi = pltpu.async_copy(iids.at[wid], iidx_v, isem.at[1])
    ldu.wait()
    ldi.wait()

    def gather(tab, idx_v, j, buf_slot, c):
        return pltpu.async_copy(
            tab.at[idx_v.at[j]], bufs.at[pl.ds(buf_slot * CH, CH)], gsem.at[c])

    g = [gather(utab, uidx_v, 0, 0, 0),
         gather(itab, iidx_v, 0, 2, 1),
         gather(utab, uidx_v, 1, 1, 2),
         gather(itab, iidx_v, 1, 3, 3),
         gather(utab, uidx_v, 2, 4, 4),
         gather(utab, uidx_v, 3, 5, 5)]

    g[0].wait()
    g[2].wait()
    wb0 = pltpu.async_copy(bufs.at[pl.ds(0, WB)], uout.at[pl.ds(base, WB)],
                           wsem.at[0])
    g[1].wait()
    g[3].wait()
    wb1 = pltpu.async_copy(bufs.at[pl.ds(2 * CH, WB)],
                           iout.at[pl.ds(base, WB)], wsem.at[1])
    g[4].wait()
    g[5].wait()
    wb2 = pltpu.async_copy(bufs.at[pl.ds(4 * CH, WB)],
                           uout.at[pl.ds(base + WB, WB)], wsem.at[2])
    wb0.wait()
    g6 = gather(itab, iidx_v, 2, 0, 6)
    g7 = gather(itab, iidx_v, 3, 1, 7)
    g6.wait()
    g7.wait()
    wb3 = pltpu.async_copy(bufs.at[pl.ds(0, WB)],
                           iout.at[pl.ds(base + WB, WB)], wsem.at[3])
    wb1.wait()
    wb2.wait()
    wb3.wait()


def kernel(user_ids, item_ids, user_table, item_table):
    uids = user_ids.astype(jnp.int32).reshape(NW, NCH, CH)
    iids = item_ids.astype(jnp.int32).reshape(NW, NCH, CH)
    return _sc_gather(uids, iids, user_table, item_table)


# speedup vs baseline: 1.0526x; 1.0526x over previous
import functools

import jax
import jax.numpy as jnp
from jax import lax
from jax.experimental import pallas as pl
from jax.experimental.pallas import tpu as pltpu
from jax.experimental.pallas import tpu_sc as plsc

BATCH = 16384
D = 128
NC = 2
NS = 16
NW = NC * NS
BPW = BATCH // NW
CH = 128
NCH = BPW // CH
NCHUNKS = 2 * NCH
NBUF = 7

_mesh = plsc.VectorSubcoreMesh(core_axis_name="c", subcore_axis_name="s")


@functools.partial(
    pl.kernel,
    mesh=_mesh,
    out_type=(
        jax.ShapeDtypeStruct((BATCH, D), jnp.float32),
        jax.ShapeDtypeStruct((BATCH, D), jnp.float32),
    ),
    scratch_types=[
        pltpu.VMEM((NCH, CH), jnp.int32),
        pltpu.VMEM((NCH, CH), jnp.int32),
        pltpu.VMEM((NBUF, CH, D), jnp.float32),
        pltpu.SemaphoreType.DMA((2,)),
        pltpu.SemaphoreType.DMA((NCHUNKS,)),
        pltpu.SemaphoreType.DMA((NCHUNKS,)),
    ],
)
def _sc_gather(uids, iids, utab, itab, uout, iout, uidx_v, iidx_v,
               bufs, isem, gsem, wsem):
    wid = lax.axis_index("s") * NC + lax.axis_index("c")
    base = wid * BPW

    ldu = pltpu.async_copy(uids.at[wid], uidx_v, isem.at[0])
    ldi = pltpu.async_copy(iids.at[wid], iidx_v, isem.at[1])
    ldu.wait()
    ldi.wait()

    def chunk(c):
        j = c // 2
        if c % 2 == 0:
            return utab, uidx_v.at[j], uout, base + j * CH
        return itab, iidx_v.at[j], iout, base + j * CH

    gathers = []
    for c in range(NCHUNKS):
        tab, idx, _, _ = chunk(c)
        if c < NBUF:
            gathers.append(pltpu.async_copy(tab.at[idx], bufs.at[c],
                                            gsem.at[c]))
        else:
            gathers.append(None)

    writebacks = []
    for c in range(NCHUNKS):
        _, _, out, off = chunk(c)
        b = c % NBUF
        gathers[c].wait()
        writebacks.append(pltpu.async_copy(bufs.at[b],
                                           out.at[pl.ds(off, CH)], wsem.at[c]))
        if c + NBUF < NCHUNKS:
            writebacks[c].wait()
            writebacks[c] = None
            tab, idx, _, _ = chunk(c + NBUF)
            gathers[c + NBUF] = pltpu.async_copy(tab.at[idx], bufs.at[b],
                                                 gsem.at[c + NBUF])

    for w in writebacks:
        if w is not None:
            w.wait()


def kernel(user_ids, item_ids, user_table, item_table):
    uids = user_ids.astype(jnp.int32).reshape(NW, NCH, CH)
    iids = item_ids.astype(jnp.int32).reshape(NW, NCH, CH)
    return _sc_gather(uids, iids, user_table, item_table)
